# baseline (speedup 1.0000x reference)
import functools

import jax
import jax.numpy as jnp
from jax import lax
from jax.experimental import pallas as pl
from jax.experimental.pallas import tpu as pltpu
from jax.experimental.pallas import tpu_sc as plsc

NC, NS, L = 2, 16, 16
NW = NC * NS
BS, C, H, W = 16, 128, 128, 128
KP = 20
K = 19
KPAD = 32
R = 2
NG = (R * W) // L
XG = W // L
ACC_STRIDE = KP * C
BPW = (BS * H // R) // NW
HHALF = H // 2


def _stage1_body(feat, labs, out, fbuf0, fbuf1, lbuf, acc, partial, invs,
                 pbase, idxv, shared, sem0, sem1, seml):
    cid = lax.axis_index("c")
    sid = lax.axis_index("s")
    wid = sid * NC + cid
    b = wid // 2
    yhalf = (wid % 2) * HHALF
    fbufs = (fbuf0, fbuf1)
    sems = (sem0, sem1)

    lab_cp = pltpu.async_copy(labs.at[b, pl.ds(yhalf, HHALF), :], lbuf, seml)

    zero = jnp.zeros((L,), jnp.float32)

    def _zero_acc(i, carry):
        for u in range(4):
            acc[pl.ds((i * 4 + u) * L, L)] = zero
        return carry
    lax.fori_loop(0, (L * ACC_STRIDE) // (L * 4), _zero_acc, 0)

    def _start_feat(t, ph):
        y0 = yhalf + t * R
        pltpu.async_copy(feat.at[b, :, pl.ds(y0, R), :], fbufs[ph], sems[ph])

    _start_feat(0, 0)
    _start_feat(1, 1)
    lab_cp.wait()

    lane_iota = lax.iota(jnp.int32, L)
    lane_off = lane_iota * ACC_STRIDE

    def _process(t, fb):
        y0l = t * R
        for g in range(NG):
            yl, xg = divmod(g, XG)
            labv = lbuf[y0l + yl, pl.ds(xg * L, L)]
            pbase[g, :] = lane_off + labv * C

        def _ss(cc, carry):
            news = []
            for g in range(NG):
                yl, xg = divmod(g, XG)
                v = fb[cc, yl, pl.ds(xg * L, L)]
                news.append(carry[g] + v * v)
            return tuple(news)
        ss = lax.fori_loop(
            0, C, _ss, tuple(jnp.zeros((L,), jnp.float32) for _ in range(NG)))

        for g in range(NG):
            s = jnp.maximum(ss[g], 1e-24)
            si = lax.bitcast_convert_type(s, jnp.int32)
            yi = jnp.int32(0x5F3759DF) - lax.shift_right_logical(si, 1)
            y = lax.bitcast_convert_type(yi, jnp.float32)
            for _ in range(3):
                y = y * (1.5 - 0.5 * s * y * y)
            invs[g, :] = y

        for g in range(NG):
            yl, xg = divmod(g, XG)
            iv = invs[g, :]
            pb = pbase[g, :]

            def _sc(cc, carry, yl=yl, xg=xg, iv=iv, pb=pb):
                v = fb[cc, yl, pl.ds(xg * L, L)] * iv
                plsc.addupdate_scatter(acc, [pb + cc], v)
                return carry
            lax.fori_loop(0, C, _sc, 0)

    def _block(i, carry):
        for ph in range(2):
            t = 2 * i + ph
            pltpu.make_async_copy(
                feat.at[b, :, pl.ds(yhalf, R), :], fbufs[ph], sems[ph]).wait()
            _process(t, fbufs[ph])

            @pl.when(t + 2 < BPW)
            def _():
                _start_feat(t + 2, ph)
        return carry
    lax.fori_loop(0, BPW // 2, _block, 0)

    def _fold(r, carry):
        for xg in range(XG):
            s = acc[pl.ds(r * C + xg * L, L)]
            for l in range(1, L):
                s = s + acc[pl.ds(l * ACC_STRIDE + r * C + xg * L, L)]
            partial[r, pl.ds(xg * L, L)] = s
        return carry
    lax.fori_loop(0, KP, _fold, 0)

    def _zpad(r, carry):
        for xg in range(XG):
            partial[r, pl.ds(xg * L, L)] = zero
        return carry
    lax.fori_loop(KP, KPAD, _zpad, 0)

    idxv[pl.ds(0, L)] = lane_iota
    idxv[pl.ds(L, L)] = lane_iota + L

    @pl.when(sid == 0)
    def _():
        pltpu.sync_copy(partial, shared)
    plsc.subcore_barrier()

    @pl.when(sid != 0)
    def _():
        pltpu.sync_copy(partial, shared.at[idxv], add=True)
    plsc.subcore_barrier()

    @pl.when(sid == 0)
    def _():
        pltpu.sync_copy(shared, out.at[cid])


_stage1 = functools.partial(
    pl.kernel,
    out_type=jax.ShapeDtypeStruct((NC, KPAD, C), jnp.float32),
    mesh=plsc.VectorSubcoreMesh(core_axis_name="c", subcore_axis_name="s"),
    compiler_params=pltpu.CompilerParams(needs_layout_passes=False),
    scratch_types=[
        pltpu.VMEM((C, R, W), jnp.float32),
        pltpu.VMEM((C, R, W), jnp.float32),
        pltpu.VMEM((HHALF, W), jnp.int32),
        pltpu.VMEM((L * ACC_STRIDE,), jnp.float32),
        pltpu.VMEM((KPAD, C), jnp.float32),
        pltpu.VMEM((NG, L), jnp.float32),
        pltpu.VMEM((NG, L), jnp.int32),
        pltpu.VMEM((KPAD,), jnp.int32),
        pltpu.VMEM_SHARED((KPAD, C), jnp.float32),
        pltpu.SemaphoreType.DMA,
        pltpu.SemaphoreType.DMA,
        pltpu.SemaphoreType.DMA,
    ],
)(_stage1_body)


def _stage2_body(p_ref, o_ref):
    a = p_ref[0] + p_ref[1]
    ss = jnp.sum(a * a, axis=1, keepdims=True)
    scaled = a / jnp.maximum(jnp.sqrt(ss), 1e-12)
    sel = (lax.broadcasted_iota(jnp.int32, (KPAD, K), 0) ==
           lax.broadcasted_iota(jnp.int32, (KPAD, K), 1)).astype(jnp.float32)
    o_ref[...] = lax.dot_general(scaled, sel, (((0,), (0,)), ((), ())),
                                 preferred_element_type=jnp.float32)


def kernel(features, labels):
    labels = labels.astype(jnp.int32)
    parts = _stage1(features, labels)
    return pl.pallas_call(
        _stage2_body,
        out_shape=jax.ShapeDtypeStruct((C, K), jnp.float32),
    )(parts)


# speedup vs baseline: 1.0019x; 1.0019x over previous
import functools

import jax
import jax.numpy as jnp
from jax import lax
from jax.experimental import pallas as pl
from jax.experimental.pallas import tpu as pltpu
from jax.experimental.pallas import tpu_sc as plsc

NC, NS, L = 2, 16, 16
NW = NC * NS
BS, C, H, W = 16, 128, 128, 128
KP = 20
K = 19
KPAD = 32
R = 2
NG = (R * W) // L
XG = W // L
ACC_STRIDE = KP * C
BPW = (BS * H // R) // NW
HHALF = H // 2


def _stage1_body(feat, labs, out, fbuf0, fbuf1, lbuf, acc, partial,
                 idxv, shared, sem0, sem1, seml):
    cid = lax.axis_index("c")
    sid = lax.axis_index("s")
    wid = sid * NC + cid
    b = wid // 2
    yhalf = (wid % 2) * HHALF
    fbufs = (fbuf0, fbuf1)
    sems = (sem0, sem1)

    lab_cp = pltpu.async_copy(labs.at[b, pl.ds(yhalf, HHALF), :], lbuf, seml)

    zero = jnp.zeros((L,), jnp.float32)

    def _zero_acc(i, carry):
        for u in range(4):
            acc[pl.ds((i * 4 + u) * L, L)] = zero
        return carry
    lax.fori_loop(0, (L * ACC_STRIDE) // (L * 4), _zero_acc, 0)

    def _start_feat(t, ph):
        y0 = yhalf + t * R
        pltpu.async_copy(feat.at[b, :, pl.ds(y0, R), :], fbufs[ph], sems[ph])

    _start_feat(0, 0)
    _start_feat(1, 1)
    lab_cp.wait()

    lane_iota = lax.iota(jnp.int32, L)
    lane_off = lane_iota * ACC_STRIDE

    def _process(t, fb):
        y0l = t * R

        def _ss(cc, carry):
            news = []
            for g in range(NG):
                yl, xg = divmod(g, XG)
                v = fb[cc, yl, pl.ds(xg * L, L)]
                news.append(carry[g] + v * v)
            return tuple(news)
        ss = lax.fori_loop(
            0, C, _ss, tuple(jnp.zeros((L,), jnp.float32) for _ in range(NG)),
            unroll=2)

        ivs, pbs = [], []
        for g in range(NG):
            s = jnp.maximum(ss[g], 1e-24)
            si = lax.bitcast_convert_type(s, jnp.int32)
            yi = jnp.int32(0x5F3759DF) - lax.shift_right_logical(si, 1)
            y = lax.bitcast_convert_type(yi, jnp.float32)
            for _ in range(3):
                y = y * (1.5 - 0.5 * s * y * y)
            ivs.append(y)
            yl, xg = divmod(g, XG)
            labv = lbuf[y0l + yl, pl.ds(xg * L, L)]
            pbs.append(lane_off + labv * C)

        def _sc(cc, carry):
            for g in range(NG):
                yl, xg = divmod(g, XG)
                v = fb[cc, yl, pl.ds(xg * L, L)] * ivs[g]
                plsc.addupdate_scatter(acc, [pbs[g] + cc], v)
            return carry
        lax.fori_loop(0, C, _sc, 0, unroll=2)

    def _block(i, carry):
        for ph in range(2):
            t = 2 * i + ph
            pltpu.make_async_copy(
                feat.at[b, :, pl.ds(yhalf, R), :], fbufs[ph], sems[ph]).wait()
            _process(t, fbufs[ph])

            @pl.when(t + 2 < BPW)
            def _():
                _start_feat(t + 2, ph)
        return carry
    lax.fori_loop(0, BPW // 2, _block, 0)

    def _fold(r, carry):
        for xg in range(XG):
            s = acc[pl.ds(r * C + xg * L, L)]
            for l in range(1, L):
                s = s + acc[pl.ds(l * ACC_STRIDE + r * C + xg * L, L)]
            partial[r, pl.ds(xg * L, L)] = s
        return carry
    lax.fori_loop(0, KP, _fold, 0)

    def _zpad(r, carry):
        for xg in range(XG):
            partial[r, pl.ds(xg * L, L)] = zero
        return carry
    lax.fori_loop(KP, KPAD, _zpad, 0)

    idxv[pl.ds(0, L)] = lane_iota
    idxv[pl.ds(L, L)] = lane_iota + L

    @pl.when(sid == 0)
    def _():
        pltpu.sync_copy(partial, shared)
    plsc.subcore_barrier()

    @pl.when(sid != 0)
    def _():
        pltpu.sync_copy(partial, shared.at[idxv], add=True)
    plsc.subcore_barrier()

    @pl.when(sid == 0)
    def _():
        pltpu.sync_copy(shared, out.at[cid])


_stage1 = functools.partial(
    pl.kernel,
    out_type=jax.ShapeDtypeStruct((NC, KPAD, C), jnp.float32),
    mesh=plsc.VectorSubcoreMesh(core_axis_name="c", subcore_axis_name="s"),
    compiler_params=pltpu.CompilerParams(needs_layout_passes=False),
    scratch_types=[
        pltpu.VMEM((C, R, W), jnp.float32),
        pltpu.VMEM((C, R, W), jnp.float32),
        pltpu.VMEM((HHALF, W), jnp.int32),
        pltpu.VMEM((L * ACC_STRIDE,), jnp.float32),
        pltpu.VMEM((KPAD, C), jnp.float32),
        pltpu.VMEM((KPAD,), jnp.int32),
        pltpu.VMEM_SHARED((KPAD, C), jnp.float32),
        pltpu.SemaphoreType.DMA,
        pltpu.SemaphoreType.DMA,
        pltpu.SemaphoreType.DMA,
    ],
)(_stage1_body)


def _stage2_body(p_ref, o_ref):
    a = p_ref[0] + p_ref[1]
    ss = jnp.sum(a * a, axis=1, keepdims=True)
    scaled = a / jnp.maximum(jnp.sqrt(ss), 1e-12)
    sel = (lax.broadcasted_iota(jnp.int32, (KPAD, K), 0) ==
           lax.broadcasted_iota(jnp.int32, (KPAD, K), 1)).astype(jnp.float32)
    o_ref[...] = lax.dot_general(scaled, sel, (((0,), (0,)), ((), ())),
                                 preferred_element_type=jnp.float32)


def kernel(features, labels):
    labels = labels.astype(jnp.int32)
    parts = _stage1(features, labels)
    return pl.pallas_call(
        _stage2_body,
        out_shape=jax.ShapeDtypeStruct((C, K), jnp.float32),
    )(parts)


# speedup vs baseline: 2.3676x; 2.3630x over previous
import functools

import jax
import jax.numpy as jnp
from jax import lax
from jax.experimental import pallas as pl
from jax.experimental.pallas import tpu as pltpu
from jax.experimental.pallas import tpu_sc as plsc

NC, NS, L = 2, 16, 16
NW = NC * NS
BS, C, H, W = 16, 128, 128, 128
KP = 20
K = 19
KPAD = 32
R = 2
NG = (R * W) // L
XG = W // L
ACC_STRIDE = KP * C
BPW = (BS * H // R) // NW
HHALF = H // 2


def _stage1_body(feat, labs, out, fbuf0, fbuf1, lbuf, acc, partial,
                 idxv, shared, sem0, sem1, seml):
    cid = lax.axis_index("c")
    sid = lax.axis_index("s")
    wid = sid * NC + cid
    b = wid // 2
    yhalf = (wid % 2) * HHALF
    fbufs = (fbuf0, fbuf1)
    sems = (sem0, sem1)

    lab_cp = pltpu.async_copy(labs.at[b, pl.ds(yhalf, HHALF), :], lbuf, seml)

    zero = jnp.zeros((L,), jnp.float32)

    def _zero_acc(i, carry):
        for u in range(4):
            acc[pl.ds((i * 4 + u) * L, L)] = zero
        return carry
    lax.fori_loop(0, (L * ACC_STRIDE) // (L * 4), _zero_acc, 0)

    def _start_feat(t, ph):
        y0 = yhalf + t * R
        pltpu.async_copy(feat.at[b, :, pl.ds(y0, R), :], fbufs[ph], sems[ph])

    _start_feat(0, 0)
    _start_feat(1, 1)
    lab_cp.wait()

    lane_iota = lax.iota(jnp.int32, L)

    def _process(t, fb):
        y0l = t * R

        def _ss(cc, carry):
            news = []
            for g in range(NG):
                yl, xg = divmod(g, XG)
                v = fb[cc, yl, pl.ds(xg * L, L)]
                news.append(carry[g] + v * v)
            return tuple(news)
        ss = lax.fori_loop(
            0, C, _ss, tuple(jnp.zeros((L,), jnp.float32) for _ in range(NG)),
            unroll=2)

        ivs, pbs = [], []
        for g in range(NG):
            s = jnp.maximum(ss[g], 1e-24)
            si = lax.bitcast_convert_type(s, jnp.int32)
            yi = jnp.int32(0x5F3759DF) - lax.shift_right_logical(si, 1)
            y = lax.bitcast_convert_type(yi, jnp.float32)
            for _ in range(3):
                y = y * (1.5 - 0.5 * s * y * y)
            ivs.append(y)
            yl, xg = divmod(g, XG)
            labv = lbuf[y0l + yl, pl.ds(xg * L, L)]
            pbs.append(labv * (C * L) + lane_iota)

        def _sc(cc, carry):
            cc16 = cc * L
            for g in range(NG):
                yl, xg = divmod(g, XG)
                v = fb[cc, yl, pl.ds(xg * L, L)] * ivs[g]
                plsc.addupdate_scatter(acc, [pbs[g] + cc16], v)
            return carry
        lax.fori_loop(0, C, _sc, 0, unroll=2)

    def _block(i, carry):
        for ph in range(2):
            t = 2 * i + ph
            pltpu.make_async_copy(
                feat.at[b, :, pl.ds(yhalf, R), :], fbufs[ph], sems[ph]).wait()
            _process(t, fbufs[ph])

            @pl.when(t + 2 < BPW)
            def _():
                _start_feat(t + 2, ph)
        return carry
    lax.fori_loop(0, BPW // 2, _block, 0)

    last_lane = lane_iota == (L - 1)

    def _fold(e, carry):
        for u in range(4):
            ee = e * 4 + u
            cum = plsc.cumsum(acc[pl.ds(ee * L, L)])
            row = jnp.broadcast_to(lax.shift_right_logical(ee, 7), (L,))
            col = jnp.broadcast_to(ee & (C - 1), (L,))
            plsc.store_scatter(partial, [row, col], cum, mask=last_lane)
        return carry
    lax.fori_loop(0, (KP * C) // 4, _fold, 0)

    def _zpad(r, carry):
        for xg in range(XG):
            partial[r, pl.ds(xg * L, L)] = zero
        return carry
    lax.fori_loop(KP, KPAD, _zpad, 0)

    idxv[pl.ds(0, L)] = lane_iota
    idxv[pl.ds(L, L)] = lane_iota + L

    @pl.when(sid == 0)
    def _():
        pltpu.sync_copy(partial, shared)
    plsc.subcore_barrier()

    @pl.when(sid != 0)
    def _():
        pltpu.sync_copy(partial, shared.at[idxv], add=True)
    plsc.subcore_barrier()

    @pl.when(sid == 0)
    def _():
        pltpu.sync_copy(shared, out.at[cid])


_stage1 = functools.partial(
    pl.kernel,
    out_type=jax.ShapeDtypeStruct((NC, KPAD, C), jnp.float32),
    mesh=plsc.VectorSubcoreMesh(core_axis_name="c", subcore_axis_name="s"),
    compiler_params=pltpu.CompilerParams(needs_layout_passes=False),
    scratch_types=[
        pltpu.VMEM((C, R, W), jnp.float32),
        pltpu.VMEM((C, R, W), jnp.float32),
        pltpu.VMEM((HHALF, W), jnp.int32),
        pltpu.VMEM((L * ACC_STRIDE,), jnp.float32),
        pltpu.VMEM((KPAD, C), jnp.float32),
        pltpu.VMEM((KPAD,), jnp.int32),
        pltpu.VMEM_SHARED((KPAD, C), jnp.float32),
        pltpu.SemaphoreType.DMA,
        pltpu.SemaphoreType.DMA,
        pltpu.SemaphoreType.DMA,
    ],
)(_stage1_body)


def _stage2_body(p_ref, o_ref):
    a = p_ref[0] + p_ref[1]
    ss = jnp.sum(a * a, axis=1, keepdims=True)
    scaled = a / jnp.maximum(jnp.sqrt(ss), 1e-12)
    sel = (lax.broadcasted_iota(jnp.int32, (KPAD, K), 0) ==
           lax.broadcasted_iota(jnp.int32, (KPAD, K), 1)).astype(jnp.float32)
    o_ref[...] = lax.dot_general(scaled, sel, (((0,), (0,)), ((), ())),
                                 preferred_element_type=jnp.float32)


def kernel(features, labels):
    labels = labels.astype(jnp.int32)
    parts = _stage1(features, labels)
    return pl.pallas_call(
        _stage2_body,
        out_shape=jax.ShapeDtypeStruct((C, K), jnp.float32),
    )(parts)
